# baseline (speedup 1.0000x reference)
import functools

import jax
import jax.numpy as jnp
import numpy as np
from jax import lax
from jax.experimental import pallas as pl
from jax.experimental.pallas import tpu as pltpu
from jax.experimental.pallas import tpu_sc as plsc

_DATA_STD = np.array([0.229, 0.224, 0.225], dtype=np.float32)
_DATA_SCALING = float(1.0 / _DATA_STD.min())

_B, _C, _D = 256, 1000, 512
_NW = 32
_RPW = _B // _NW
_L = 16
_NFULL = _C // _L

_GDN = lax.GatherDimensionNumbers(
    offset_dims=(), collapsed_slice_dims=(0,), start_index_map=(0,)
)


def _vperm(x, idx):
    return lax.gather(
        x, idx[:, None], _GDN, (1,),
        mode=lax.GatherScatterMode.PROMISE_IN_BOUNDS,
    )


def _sc_body(pred_hbm, jpad_hbm, *refs):
    bufs, jv, sem = refs[:_RPW], refs[_RPW], refs[_RPW + 1]
    info = plsc.get_sparse_core_info()
    nc = info.num_cores
    wid = lax.axis_index("s") * nc + lax.axis_index("c")
    base = wid * _RPW

    cps = [
        pltpu.async_copy(
            pred_hbm.at[pl.ds((base + r) * _C, _C)], bufs[r], sem
        )
        for r in range(_RPW)
    ]
    for cp in cps:
        cp.wait()

    lane = lax.broadcasted_iota(jnp.int32, (_L,), 0)

    def chunk(i, carry):
        idx = i * _L + lane
        out = []
        for r in range(_RPW):
            m, bi = carry[2 * r], carry[2 * r + 1]
            v = bufs[r][pl.ds(i * _L, _L)]
            upd = v > m
            out.append(jnp.where(upd, v, m))
            out.append(jnp.where(upd, idx, bi))
        return tuple(out)

    m0 = jnp.full((_L,), -jnp.inf, jnp.float32)
    init = (m0, jnp.zeros((_L,), jnp.int32)) * _RPW
    carry = lax.fori_loop(0, _NFULL, chunk, init, unroll=2)

    jvec = jnp.zeros((_L,), jnp.int32)
    tidx = (_C - _L) + lane
    for r in range(_RPW):
        m, bi = carry[2 * r], carry[2 * r + 1]
        tv = bufs[r][pl.ds(_C - _L, _L)]
        upd = tv > m
        m = jnp.where(upd, tv, m)
        bi = jnp.where(upd, tidx, bi)
        for s in (8, 4, 2, 1):
            perm = jnp.bitwise_xor(lane, s)
            pm = _vperm(m, perm)
            pb = _vperm(bi, perm)
            take = (pm > m) | ((pm == m) & (pb < bi))
            m = jnp.where(take, pm, m)
            bi = jnp.where(take, pb, bi)
        jvec = jnp.where(lane == r, bi, jvec)

    jv[...] = jvec
    pltpu.sync_copy(jv, jpad_hbm.at[wid])


@functools.cache
def _make_sc_call():
    return functools.partial(
        pl.kernel,
        mesh=plsc.VectorSubcoreMesh(core_axis_name="c", subcore_axis_name="s"),
        out_type=jax.ShapeDtypeStruct((_NW, _L), jnp.int32),
        scratch_types=[pltpu.VMEM((_C,), jnp.float32)] * _RPW + [
            pltpu.VMEM((_L,), jnp.int32),
            pltpu.SemaphoreType.DMA,
        ],
    )(_sc_body)


def _tc_body(pred_ref, w_ref, j_ref, k_ref, out_ref):
    W = w_ref[...]
    pred = pred_ref[...]
    j = j_ref[...]
    K = k_ref[0, 0]

    colid = lax.broadcasted_iota(jnp.int32, (_B, _C), 1)
    onehot = jnp.where(colid == j, 1.0, 0.0)
    Wj = lax.dot_general(onehot, W, (((1,), (0,)), ((), ())),
                         preferred_element_type=jnp.float32)

    inv_n = lax.rsqrt(jnp.sum(W * W, axis=1))
    inv_nj = lax.rsqrt(jnp.sum(Wj * Wj, axis=1, keepdims=True))
    S = lax.dot_general(Wj, W, (((1,), (1,)), ((), ())),
                        preferred_element_type=jnp.float32)
    cos = S * inv_n[None, :] * inv_nj
    kij = K * jnp.sqrt(jnp.maximum(2.0 - 2.0 * cos, 0.0))

    y = jnp.max(pred, axis=1, keepdims=True)
    margins = y - pred
    ratios = jnp.where(colid == j, jnp.inf, margins / kij)
    rmin = jnp.min(ratios, axis=1)
    out_ref[0, 0] = jnp.sum(rmin) * (1.0 / _B)


def _tc_call(pred, W, j, k):
    return pl.pallas_call(
        _tc_body,
        out_shape=jax.ShapeDtypeStruct((1, 1), jnp.float32),
        in_specs=[
            pl.BlockSpec(memory_space=pltpu.VMEM),
            pl.BlockSpec(memory_space=pltpu.VMEM),
            pl.BlockSpec(memory_space=pltpu.VMEM),
            pl.BlockSpec(memory_space=pltpu.SMEM),
        ],
        out_specs=pl.BlockSpec(memory_space=pltpu.SMEM),
    )(pred, W, j, k)


def kernel(prediction, target, W, K_model, Kfc):
    k = (K_model / Kfc * _DATA_SCALING).astype(jnp.float32).reshape(1, 1)
    jpad = _make_sc_call()(prediction.reshape(-1))
    j = jpad[:, :_RPW].reshape(_B, 1)
    out = _tc_call(prediction, W, j, k)
    return out[0, 0]


# speedup vs baseline: 1.0058x; 1.0058x over previous
import functools

import jax
import jax.numpy as jnp
import numpy as np
from jax import lax
from jax.experimental import pallas as pl
from jax.experimental.pallas import tpu as pltpu
from jax.experimental.pallas import tpu_sc as plsc

_DATA_STD = np.array([0.229, 0.224, 0.225], dtype=np.float32)
_DATA_SCALING = float(1.0 / _DATA_STD.min())

_B, _C, _D = 256, 1000, 512
_NW = 32
_RPW = _B // _NW
_L = 16
_NFULL = _C // _L

_GDN = lax.GatherDimensionNumbers(
    offset_dims=(), collapsed_slice_dims=(0,), start_index_map=(0,)
)


def _vperm(x, idx):
    return lax.gather(
        x, idx[:, None], _GDN, (1,),
        mode=lax.GatherScatterMode.PROMISE_IN_BOUNDS,
    )


def _sc_body(pred_hbm, jpad_hbm, *refs):
    bufs, jv, sem = refs[:_RPW], refs[_RPW], refs[_RPW + 1]
    info = plsc.get_sparse_core_info()
    nc = info.num_cores
    wid = lax.axis_index("s") * nc + lax.axis_index("c")
    base = wid * _RPW

    cps = [
        pltpu.async_copy(
            pred_hbm.at[pl.ds((base + r) * _C, _C)], bufs[r], sem
        )
        for r in range(_RPW)
    ]
    for cp in cps:
        cp.wait()

    lane = lax.broadcasted_iota(jnp.int32, (_L,), 0)

    def chunk(i, carry):
        idx = i * _L + lane
        out = []
        for r in range(_RPW):
            m, bi = carry[2 * r], carry[2 * r + 1]
            v = bufs[r][pl.ds(i * _L, _L)]
            upd = v > m
            out.append(jnp.where(upd, v, m))
            out.append(jnp.where(upd, idx, bi))
        return tuple(out)

    m0 = jnp.full((_L,), -jnp.inf, jnp.float32)
    init = (m0, jnp.zeros((_L,), jnp.int32)) * _RPW
    carry = lax.fori_loop(0, _NFULL, chunk, init, unroll=2)

    jvec = jnp.zeros((_L,), jnp.int32)
    tidx = (_C - _L) + lane
    for r in range(_RPW):
        m, bi = carry[2 * r], carry[2 * r + 1]
        tv = bufs[r][pl.ds(_C - _L, _L)]
        upd = tv > m
        m = jnp.where(upd, tv, m)
        bi = jnp.where(upd, tidx, bi)
        for s in (8, 4, 2, 1):
            perm = jnp.bitwise_xor(lane, s)
            pm = _vperm(m, perm)
            pb = _vperm(bi, perm)
            take = (pm > m) | ((pm == m) & (pb < bi))
            m = jnp.where(take, pm, m)
            bi = jnp.where(take, pb, bi)
        jvec = jnp.where(lane == r, bi, jvec)

    jv[...] = jvec
    pltpu.sync_copy(jv.at[pl.ds(0, _RPW)], jpad_hbm.at[pl.ds(base, _RPW)])


@functools.cache
def _make_sc_call():
    return functools.partial(
        pl.kernel,
        mesh=plsc.VectorSubcoreMesh(core_axis_name="c", subcore_axis_name="s"),
        out_type=jax.ShapeDtypeStruct((_B,), jnp.int32),
        scratch_types=[pltpu.VMEM((_C,), jnp.float32)] * _RPW + [
            pltpu.VMEM((_L,), jnp.int32),
            pltpu.SemaphoreType.DMA,
        ],
    )(_sc_body)


def _tc_body(pred_ref, w_ref, j_ref, k_ref, out_ref):
    W = w_ref[...]
    pred = pred_ref[...]
    j = j_ref[...]
    K = k_ref[0, 0]

    colid = lax.broadcasted_iota(jnp.int32, (_B, _C), 1)
    onehot = jnp.where(colid == j, 1.0, 0.0)
    Wj = lax.dot_general(onehot, W, (((1,), (0,)), ((), ())),
                         preferred_element_type=jnp.float32)

    inv_n = lax.rsqrt(jnp.sum(W * W, axis=1))
    inv_nj = lax.rsqrt(jnp.sum(Wj * Wj, axis=1, keepdims=True))
    S = lax.dot_general(Wj, W, (((1,), (1,)), ((), ())),
                        preferred_element_type=jnp.float32)
    cos = S * inv_n[None, :] * inv_nj
    kij = K * jnp.sqrt(jnp.maximum(2.0 - 2.0 * cos, 0.0))

    y = jnp.max(pred, axis=1, keepdims=True)
    margins = y - pred
    ratios = jnp.where(colid == j, jnp.inf, margins / kij)
    rmin = jnp.min(ratios, axis=1)
    out_ref[0, 0] = jnp.sum(rmin) * (1.0 / _B)


def _tc_call(pred, W, j, k):
    return pl.pallas_call(
        _tc_body,
        out_shape=jax.ShapeDtypeStruct((1, 1), jnp.float32),
        in_specs=[
            pl.BlockSpec(memory_space=pltpu.VMEM),
            pl.BlockSpec(memory_space=pltpu.VMEM),
            pl.BlockSpec(memory_space=pltpu.VMEM),
            pl.BlockSpec(memory_space=pltpu.SMEM),
        ],
        out_specs=pl.BlockSpec(memory_space=pltpu.SMEM),
    )(pred, W, j, k)


def kernel(prediction, target, W, K_model, Kfc):
    k = (K_model / Kfc * _DATA_SCALING).astype(jnp.float32).reshape(1, 1)
    jflat = _make_sc_call()(prediction.reshape(-1))
    out = _tc_call(prediction, W, jflat.reshape(_B, 1), k)
    return out[0, 0]


# speedup vs baseline: 1.0346x; 1.0286x over previous
import functools

import jax
import jax.numpy as jnp
import numpy as np
from jax import lax
from jax.experimental import pallas as pl
from jax.experimental.pallas import tpu as pltpu
from jax.experimental.pallas import tpu_sc as plsc

_DATA_STD = np.array([0.229, 0.224, 0.225], dtype=np.float32)
_DATA_SCALING = float(1.0 / _DATA_STD.min())

_B, _C, _D = 256, 1000, 512
_NW = 32
_RPW = _B // _NW
_L = 16
_NFULL = _C // _L

_GDN = lax.GatherDimensionNumbers(
    offset_dims=(), collapsed_slice_dims=(0,), start_index_map=(0,)
)


def _vperm(x, idx):
    return lax.gather(
        x, idx[:, None], _GDN, (1,),
        mode=lax.GatherScatterMode.PROMISE_IN_BOUNDS,
    )


def _sc_body(pred_hbm, jpad_hbm, *refs):
    bufs, jv, sem = refs[:_RPW], refs[_RPW], refs[_RPW + 1]
    info = plsc.get_sparse_core_info()
    nc = info.num_cores
    wid = lax.axis_index("s") * nc + lax.axis_index("c")
    base = wid * _RPW

    cps = [
        pltpu.async_copy(pred_hbm.at[base + r], bufs[r], sem)
        for r in range(_RPW)
    ]
    for cp in cps:
        cp.wait()

    lane = lax.broadcasted_iota(jnp.int32, (_L,), 0)

    def chunk(i, carry):
        idx = i * _L + lane
        out = []
        for r in range(_RPW):
            m, bi = carry[2 * r], carry[2 * r + 1]
            v = bufs[r][pl.ds(i * _L, _L)]
            upd = v > m
            out.append(jnp.where(upd, v, m))
            out.append(jnp.where(upd, idx, bi))
        return tuple(out)

    m0 = jnp.full((_L,), -jnp.inf, jnp.float32)
    init = (m0, jnp.zeros((_L,), jnp.int32)) * _RPW
    carry = lax.fori_loop(0, _NFULL, chunk, init, unroll=2)

    jvec = jnp.zeros((_L,), jnp.int32)
    tidx = (_C - _L) + lane
    for r in range(_RPW):
        m, bi = carry[2 * r], carry[2 * r + 1]
        tv = bufs[r][pl.ds(_C - _L, _L)]
        upd = tv > m
        m = jnp.where(upd, tv, m)
        bi = jnp.where(upd, tidx, bi)
        for s in (8, 4, 2, 1):
            perm = jnp.bitwise_xor(lane, s)
            pm = _vperm(m, perm)
            pb = _vperm(bi, perm)
            take = (pm > m) | ((pm == m) & (pb < bi))
            m = jnp.where(take, pm, m)
            bi = jnp.where(take, pb, bi)
        jvec = jnp.where(lane == r, bi, jvec)

    jv[...] = jvec
    pltpu.sync_copy(jv.at[pl.ds(0, _RPW)], jpad_hbm.at[pl.ds(base, _RPW)])


@functools.cache
def _make_sc_call():
    return functools.partial(
        pl.kernel,
        mesh=plsc.VectorSubcoreMesh(core_axis_name="c", subcore_axis_name="s"),
        out_type=jax.ShapeDtypeStruct((_B,), jnp.int32),
        scratch_types=[pltpu.VMEM((_C,), jnp.float32)] * _RPW + [
            pltpu.VMEM((_L,), jnp.int32),
            pltpu.SemaphoreType.DMA,
        ],
    )(_sc_body)


def _tc_body(pred_ref, w_ref, j_ref, k_ref, out_ref):
    W = w_ref[...]
    pred = pred_ref[...]
    j = j_ref[...]
    K = k_ref[0, 0]

    colid = lax.broadcasted_iota(jnp.int32, (_B, _C), 1)
    onehot = jnp.where(colid == j, 1.0, 0.0)
    Wj = lax.dot_general(onehot, W, (((1,), (0,)), ((), ())),
                         preferred_element_type=jnp.float32)

    inv_n = lax.rsqrt(jnp.sum(W * W, axis=1))
    inv_nj = lax.rsqrt(jnp.sum(Wj * Wj, axis=1, keepdims=True))
    S = lax.dot_general(Wj, W, (((1,), (1,)), ((), ())),
                        preferred_element_type=jnp.float32)
    cos = S * inv_n[None, :] * inv_nj
    kij = K * jnp.sqrt(jnp.maximum(2.0 - 2.0 * cos, 0.0))

    y = jnp.max(pred, axis=1, keepdims=True)
    margins = y - pred
    ratios = jnp.where(colid == j, jnp.inf, margins / kij)
    rmin = jnp.min(ratios, axis=1)
    out_ref[0, 0] = jnp.sum(rmin) * (1.0 / _B)


def _tc_call(pred, W, j, k):
    return pl.pallas_call(
        _tc_body,
        out_shape=jax.ShapeDtypeStruct((1, 1), jnp.float32),
        in_specs=[
            pl.BlockSpec(memory_space=pltpu.VMEM),
            pl.BlockSpec(memory_space=pltpu.VMEM),
            pl.BlockSpec(memory_space=pltpu.VMEM),
            pl.BlockSpec(memory_space=pltpu.SMEM),
        ],
        out_specs=pl.BlockSpec(memory_space=pltpu.SMEM),
    )(pred, W, j, k)


def kernel(prediction, target, W, K_model, Kfc):
    k = (K_model / Kfc * _DATA_SCALING).astype(jnp.float32).reshape(1, 1)
    jflat = _make_sc_call()(prediction)
    out = _tc_call(prediction, W, jflat.reshape(_B, 1), k)
    return out[0, 0]


# speedup vs baseline: 1.0951x; 1.0585x over previous
import functools

import jax
import jax.numpy as jnp
import numpy as np
from jax import lax
from jax.experimental import pallas as pl
from jax.experimental.pallas import tpu as pltpu
from jax.experimental.pallas import tpu_sc as plsc

_DATA_STD = np.array([0.229, 0.224, 0.225], dtype=np.float32)
_DATA_SCALING = float(1.0 / _DATA_STD.min())

_B, _C, _D = 256, 1000, 512
_NW = 32
_RPW = _B // _NW
_L = 16
_NFULL = _C // _L

_GDN = lax.GatherDimensionNumbers(
    offset_dims=(), collapsed_slice_dims=(0,), start_index_map=(0,)
)


def _vperm(x, idx):
    return lax.gather(
        x, idx[:, None], _GDN, (1,),
        mode=lax.GatherScatterMode.PROMISE_IN_BOUNDS,
    )


def _sc_body(pred_hbm, jpad_hbm, *refs):
    bufs, jv, sem = refs[:_RPW], refs[_RPW], refs[_RPW + 1]
    info = plsc.get_sparse_core_info()
    nc = info.num_cores
    wid = lax.axis_index("s") * nc + lax.axis_index("c")
    base = wid * _RPW

    cps = [
        pltpu.async_copy(pred_hbm.at[base + r], bufs[r], sem)
        for r in range(_RPW)
    ]
    for cp in cps:
        cp.wait()

    lane = lax.broadcasted_iota(jnp.int32, (_L,), 0)

    def chunk(i, carry):
        idx = i * _L + lane
        out = []
        for r in range(_RPW):
            m, bi = carry[2 * r], carry[2 * r + 1]
            v = bufs[r][pl.ds(i * _L, _L)]
            upd = v > m
            out.append(jnp.where(upd, v, m))
            out.append(jnp.where(upd, idx, bi))
        return tuple(out)

    m0 = jnp.full((_L,), -jnp.inf, jnp.float32)
    init = (m0, jnp.zeros((_L,), jnp.int32)) * _RPW
    carry = lax.fori_loop(0, _NFULL, chunk, init, unroll=2)

    jvec = jnp.zeros((_L,), jnp.int32)
    tidx = (_C - _L) + lane
    for r in range(_RPW):
        m, bi = carry[2 * r], carry[2 * r + 1]
        tv = bufs[r][pl.ds(_C - _L, _L)]
        upd = tv > m
        m = jnp.where(upd, tv, m)
        bi = jnp.where(upd, tidx, bi)
        for s in (8, 4, 2, 1):
            perm = jnp.bitwise_xor(lane, s)
            pm = _vperm(m, perm)
            pb = _vperm(bi, perm)
            take = (pm > m) | ((pm == m) & (pb < bi))
            m = jnp.where(take, pm, m)
            bi = jnp.where(take, pb, bi)
        jvec = jnp.where(lane == r, bi, jvec)

    jv[...] = jvec
    pltpu.sync_copy(jv.at[pl.ds(0, _RPW)], jpad_hbm.at[pl.ds(base, _RPW)])


@functools.cache
def _make_sc_call():
    return functools.partial(
        pl.kernel,
        mesh=plsc.VectorSubcoreMesh(core_axis_name="c", subcore_axis_name="s"),
        out_type=jax.ShapeDtypeStruct((_B,), jnp.int32),
        scratch_types=[pltpu.VMEM((_C,), jnp.float32)] * _RPW + [
            pltpu.VMEM((_L,), jnp.int32),
            pltpu.SemaphoreType.DMA,
        ],
    )(_sc_body)


def _tc_body(pred_ref, w_ref, j_ref, k_ref, out_ref):
    W = w_ref[...]
    pred = pred_ref[...]
    j = lax.transpose(j_ref[...], (1, 0))
    K = k_ref[0, 0]

    colid = lax.broadcasted_iota(jnp.int32, (_B, _C), 1)
    onehot = jnp.where(colid == j, 1.0, 0.0)
    Wj = lax.dot_general(onehot, W, (((1,), (0,)), ((), ())),
                         preferred_element_type=jnp.float32)

    inv_n = lax.rsqrt(jnp.sum(W * W, axis=1))
    inv_nj = lax.rsqrt(jnp.sum(Wj * Wj, axis=1, keepdims=True))
    S = lax.dot_general(Wj, W, (((1,), (1,)), ((), ())),
                        preferred_element_type=jnp.float32)
    cos = S * inv_n[None, :] * inv_nj
    kij = K * jnp.sqrt(jnp.maximum(2.0 - 2.0 * cos, 0.0))

    y = jnp.max(pred, axis=1, keepdims=True)
    margins = y - pred
    ratios = jnp.where(colid == j, jnp.inf, margins / kij)
    rmin = jnp.min(ratios, axis=1)
    out_ref[0, 0] = jnp.sum(rmin) * (1.0 / _B)


def _tc_call(pred, W, j, k):
    return pl.pallas_call(
        _tc_body,
        out_shape=jax.ShapeDtypeStruct((1, 1), jnp.float32),
        in_specs=[
            pl.BlockSpec(memory_space=pltpu.VMEM),
            pl.BlockSpec(memory_space=pltpu.VMEM),
            pl.BlockSpec(memory_space=pltpu.VMEM),
            pl.BlockSpec(memory_space=pltpu.SMEM),
        ],
        out_specs=pl.BlockSpec(memory_space=pltpu.SMEM),
    )(pred, W, j, k)


def kernel(prediction, target, W, K_model, Kfc):
    k = (K_model / Kfc * _DATA_SCALING).astype(jnp.float32).reshape(1, 1)
    jflat = _make_sc_call()(prediction)
    out = _tc_call(prediction, W, jflat.reshape(1, _B), k)
    return out[0, 0]
